# baseline (speedup 1.0000x reference)
import jax
import jax.numpy as jnp
from jax.experimental import pallas as pl
from jax.experimental.pallas import tpu as pltpu

NUM_EXPERTS = 16
HIDDEN_DIM = 1024
EXPERT_DIM = 512
T = 32
TOP_K = 2
NBUF = 4


def _moe_kernel(idx_ref, scores_ref, x_ref, up_hbm, dn_hbm, out_ref,
                up_buf, dn_buf, up_sem, dn_sem):
    def issue(e):
        slot = e % NBUF
        pltpu.make_async_copy(up_hbm.at[e], up_buf.at[slot],
                              up_sem.at[slot]).start()
        pltpu.make_async_copy(dn_hbm.at[e], dn_buf.at[slot],
                              dn_sem.at[slot]).start()

    for e in range(NBUF - 1):
        issue(e)

    x = x_ref[...]
    acc = jnp.zeros((T, HIDDEN_DIM), jnp.float32)
    for e in range(NUM_EXPERTS):
        slot = e % NBUF
        pltpu.make_async_copy(up_hbm.at[e], up_buf.at[slot],
                              up_sem.at[slot]).wait()
        pltpu.make_async_copy(dn_hbm.at[e], dn_buf.at[slot],
                              dn_sem.at[slot]).wait()
        if e + NBUF - 1 < NUM_EXPERTS:
            issue(e + NBUF - 1)

        mask = (idx_ref[...] == e).astype(jnp.float32)
        w = jnp.sum(scores_ref[...] * mask, axis=1)

        h = jnp.dot(x, up_buf[slot], preferred_element_type=jnp.float32)
        g = h[:, :EXPERT_DIM]
        u = h[:, EXPERT_DIM:]
        y = (g * jax.nn.sigmoid(g)) * u
        o = jnp.dot(y, dn_buf[slot], preferred_element_type=jnp.float32)
        acc = acc + o * w[:, None]

    out_ref[...] = acc


@jax.jit
def kernel(x, expert_indices, scores, up_proj, down_proj):
    return pl.pallas_call(
        _moe_kernel,
        in_specs=[
            pl.BlockSpec(memory_space=pltpu.MemorySpace.VMEM),
            pl.BlockSpec(memory_space=pltpu.MemorySpace.VMEM),
            pl.BlockSpec(memory_space=pltpu.MemorySpace.VMEM),
            pl.BlockSpec(memory_space=pl.ANY),
            pl.BlockSpec(memory_space=pl.ANY),
        ],
        out_specs=pl.BlockSpec(memory_space=pltpu.MemorySpace.VMEM),
        out_shape=jax.ShapeDtypeStruct((T, HIDDEN_DIM), jnp.float32),
        scratch_shapes=[
            pltpu.VMEM((NBUF, HIDDEN_DIM, 2 * EXPERT_DIM), jnp.float32),
            pltpu.VMEM((NBUF, EXPERT_DIM, HIDDEN_DIM), jnp.float32),
            pltpu.SemaphoreType.DMA((NBUF,)),
            pltpu.SemaphoreType.DMA((NBUF,)),
        ],
    )(expert_indices, scores, x, up_proj, down_proj)


# speedup vs baseline: 1.0788x; 1.0788x over previous
import jax
import jax.numpy as jnp
from jax.experimental import pallas as pl
from jax.experimental.pallas import tpu as pltpu

NUM_EXPERTS = 16
HIDDEN_DIM = 1024
EXPERT_DIM = 512
T = 32
TOP_K = 2


def _moe_kernel(idx_ref, scores_ref, x_ref, u0_ref, u1_ref, dn_ref, out_ref):
    e = pl.program_id(0)
    mask = (idx_ref[...] == e).astype(jnp.float32)
    w = jnp.sum(scores_ref[...] * mask, axis=1)

    x = x_ref[...]
    half = HIDDEN_DIM // 2
    h = (jnp.dot(x[:, :half], u0_ref[0], preferred_element_type=jnp.float32)
         + jnp.dot(x[:, half:], u1_ref[0], preferred_element_type=jnp.float32))
    g = h[:, :EXPERT_DIM]
    u = h[:, EXPERT_DIM:]
    y = (g * jax.nn.sigmoid(g)) * u
    o = jnp.dot(y, dn_ref[0], preferred_element_type=jnp.float32)
    contrib = o * w[:, None]

    @pl.when(e == 0)
    def _init():
        out_ref[...] = contrib

    @pl.when(e != 0)
    def _acc():
        out_ref[...] += contrib


@jax.jit
def kernel(x, expert_indices, scores, up_proj, down_proj):
    grid = (NUM_EXPERTS,)
    return pl.pallas_call(
        _moe_kernel,
        grid=grid,
        in_specs=[
            pl.BlockSpec((T, TOP_K), lambda e: (0, 0)),
            pl.BlockSpec((T, TOP_K), lambda e: (0, 0)),
            pl.BlockSpec((T, HIDDEN_DIM), lambda e: (0, 0)),
            pl.BlockSpec((1, HIDDEN_DIM // 2, 2 * EXPERT_DIM),
                         lambda e: (e, 0, 0)),
            pl.BlockSpec((1, HIDDEN_DIM // 2, 2 * EXPERT_DIM),
                         lambda e: (e, 1, 0)),
            pl.BlockSpec((1, EXPERT_DIM, HIDDEN_DIM), lambda e: (e, 0, 0)),
        ],
        out_specs=pl.BlockSpec((T, HIDDEN_DIM), lambda e: (0, 0)),
        out_shape=jax.ShapeDtypeStruct((T, HIDDEN_DIM), jnp.float32),
        compiler_params=pltpu.CompilerParams(
            dimension_semantics=("arbitrary",),
        ),
    )(expert_indices, scores, x, up_proj, up_proj, down_proj)
